# baseline (speedup 1.0000x reference)
import functools

import jax
import jax.numpy as jnp
from jax import lax
from jax.experimental import pallas as pl
from jax.experimental.pallas import tpu as pltpu
from jax.experimental.pallas import tpu_sc as plsc

B = 4096
T = 50
CONT = 16
NCAT = 26
V = 100000
ED = 32
ZD = 1024
IN_DIM = CONT + NCAT * ED
VPAD = 896

NC = 2
NS = 16
NW = NC * NS
BPW = B // NW
GRP = 8
NGRP = BPW // GRP

NIDS = T * NCAT
NSTREAM = 11
SLEN = 120
IDS_PAD = NSTREAM * SLEN

UNROLL = 10


def _pool_body(ids_hbm, tab_hbm, v_hbm, idx_g, rows0, rows1, out_v, sem0, sem1):
    wid = lax.axis_index("s") * NC + lax.axis_index("c")
    base = wid * BPW

    zeros32 = jnp.zeros((32,), jnp.bfloat16)
    for bi in range(GRP):
        out_v[bi, pl.ds(0, 32)] = zeros32
        out_v[bi, pl.ds(832, 32)] = zeros32
        out_v[bi, pl.ds(864, 32)] = zeros32

    def fire(bi, rows_v, sem):
        for j in range(NSTREAM):
            pltpu.async_copy(
                tab_hbm.at[idx_g.at[bi, j]],
                rows_v.at[pl.ds(j * SLEN, SLEN)],
                sem,
            )

    def wait(bi, rows_v, sem):
        for j in range(NSTREAM):
            pltpu.make_async_copy(
                tab_hbm.at[idx_g.at[bi, j]],
                rows_v.at[pl.ds(j * SLEN, SLEN)],
                sem,
            ).wait()

    def accum(bi, rows_v):
        def field(f, carry):
            def cat_step(t5, acc):
                r0 = t5 * (UNROLL * NCAT) + f
                for u in range(UNROLL):
                    acc = acc + rows_v[r0 + u * NCAT, :]
                return acc

            acc = lax.fori_loop(0, T // UNROLL, cat_step, zeros32)
            out_v[bi, pl.ds(CONT + f * ED, 32)] = acc
            return carry

        lax.fori_loop(0, NCAT, field, 0)

    def body(g, carry):
        b0 = base + g * GRP
        pltpu.sync_copy(ids_hbm.at[pl.ds(b0, GRP)], idx_g)
        fire(0, rows0, sem0)
        for bi in range(1, GRP + 1):
            if bi < GRP:
                fire(bi, (rows0, rows1)[bi % 2], (sem0, sem1)[bi % 2])
            wait(bi - 1, (rows0, rows1)[(bi - 1) % 2], (sem0, sem1)[(bi - 1) % 2])
            accum(bi - 1, (rows0, rows1)[(bi - 1) % 2])
        pltpu.sync_copy(out_v, v_hbm.at[pl.ds(b0, GRP)])
        return carry

    lax.fori_loop(0, NGRP, body, 0)


_pool = pl.kernel(
    _pool_body,
    out_type=jax.ShapeDtypeStruct((B, VPAD), jnp.bfloat16),
    mesh=plsc.VectorSubcoreMesh(
        core_axis_name="c", subcore_axis_name="s", num_cores=NC, num_subcores=NS
    ),
    scratch_types=[
        pltpu.VMEM((GRP, NSTREAM, SLEN), jnp.int32),
        pltpu.VMEM((IDS_PAD, ED), jnp.bfloat16),
        pltpu.VMEM((IDS_PAD, ED), jnp.bfloat16),
        pltpu.VMEM((GRP, VPAD), jnp.bfloat16),
        pltpu.SemaphoreType.DMA,
        pltpu.SemaphoreType.DMA,
    ],
    compiler_params=pltpu.CompilerParams(use_tc_tiling_on_sc=False),
)

BB = 512


def _mix_body(z_ref, v_ref, c_ref, m_ref, wp_ref, wc_ref, bp_ref, wg_ref, bg_ref, o_ref):
    zb = z_ref[...]
    gate = jax.nn.sigmoid(
        jnp.dot(zb, wg_ref[...], preferred_element_type=jnp.float32) + bg_ref[...]
    )
    cpool = jnp.dot(c_ref[...], m_ref[...], preferred_element_type=jnp.float32)
    v32 = v_ref[...].astype(jnp.float32)
    exo = (
        jnp.dot(v32, wp_ref[...], preferred_element_type=jnp.float32)
        + jnp.dot(cpool, wc_ref[...], preferred_element_type=jnp.float32)
        + bp_ref[...]
    )
    o_ref[...] = zb + gate * exo


def _mix(z, v, cont2, m, wp, wc, bp, wg, bg):
    return pl.pallas_call(
        _mix_body,
        grid=(B // BB,),
        in_specs=[
            pl.BlockSpec((BB, ZD), lambda i: (i, 0)),
            pl.BlockSpec((BB, VPAD), lambda i: (i, 0)),
            pl.BlockSpec((BB, T * CONT), lambda i: (i, 0)),
            pl.BlockSpec((T * CONT, CONT), lambda i: (0, 0)),
            pl.BlockSpec((VPAD, ZD), lambda i: (0, 0)),
            pl.BlockSpec((CONT, ZD), lambda i: (0, 0)),
            pl.BlockSpec((1, ZD), lambda i: (0, 0)),
            pl.BlockSpec((ZD, ZD), lambda i: (0, 0)),
            pl.BlockSpec((1, ZD), lambda i: (0, 0)),
        ],
        out_specs=pl.BlockSpec((BB, ZD), lambda i: (i, 0)),
        out_shape=jax.ShapeDtypeStruct((B, ZD), jnp.float32),
    )(z, v, cont2, m, wp, wc, bp, wg, bg)


def kernel(z, past_exo_cont, past_exo_cat, tables, W_proj, b_proj, W_gate, b_gate):
    ids = jnp.clip(past_exo_cat, 0, V - 1).astype(jnp.int32)
    off = jnp.arange(NCAT, dtype=jnp.int32) * V
    flat = (ids + off[None, None, :]).reshape(B, NIDS)
    flat = jnp.pad(flat, ((0, 0), (0, IDS_PAD - NIDS)))
    flat = flat.reshape(B, NSTREAM, SLEN)
    tab = tables.reshape(NCAT * V, ED).astype(jnp.bfloat16)

    v = _pool(flat, tab)

    cont2 = past_exo_cont.reshape(B, T * CONT)
    m = jnp.tile(jnp.eye(CONT, dtype=jnp.float32), (T, 1)) * (1.0 / T)
    wp = jnp.concatenate(
        [jnp.zeros((CONT, ZD), W_proj.dtype), W_proj[CONT:] * (1.0 / T),
         jnp.zeros((VPAD - IN_DIM, ZD), W_proj.dtype)], axis=0
    )
    return _mix(
        z, v, cont2, m, wp, W_proj[:CONT],
        b_proj.reshape(1, ZD), W_gate, b_gate.reshape(1, ZD),
    )


# speedup vs baseline: 1.1352x; 1.1352x over previous
import functools

import jax
import jax.numpy as jnp
from jax import lax
from jax.experimental import pallas as pl
from jax.experimental.pallas import tpu as pltpu
from jax.experimental.pallas import tpu_sc as plsc

B = 4096
T = 50
CONT = 16
NCAT = 26
V = 100000
ED = 32
ZD = 1024
IN_DIM = CONT + NCAT * ED
VPAD = 896

NC = 2
NS = 16
NW = NC * NS
BPW = B // NW
GRP = 8
NGRP = BPW // GRP

NIDS = T * NCAT

UNROLL = 10


def _pool_body(ids_hbm, tab_hbm, v_hbm, idx_g, rows0, rows1, out_v, sem0, sem1):
    wid = lax.axis_index("s") * NC + lax.axis_index("c")
    base = wid * BPW

    zeros32 = jnp.zeros((32,), jnp.bfloat16)
    for bi in range(GRP):
        out_v[bi, pl.ds(0, 32)] = zeros32
        out_v[bi, pl.ds(832, 32)] = zeros32
        out_v[bi, pl.ds(864, 32)] = zeros32

    def fire(bi, rows_v, sem):
        for f in range(NCAT):
            pltpu.async_copy(
                tab_hbm.at[f].at[idx_g.at[bi, f]],
                rows_v.at[pl.ds(f * T, T)],
                sem,
            )

    def wait(bi, rows_v, sem):
        for f in range(NCAT):
            pltpu.make_async_copy(
                tab_hbm.at[f].at[idx_g.at[bi, f]],
                rows_v.at[pl.ds(f * T, T)],
                sem,
            ).wait()

    def accum(bi, rows_v):
        def field(f, carry):
            def cat_step(t5, acc):
                r0 = f * T + t5 * UNROLL
                for u in range(UNROLL):
                    acc = acc + rows_v[r0 + u, :]
                return acc

            acc = lax.fori_loop(0, T // UNROLL, cat_step, zeros32)
            out_v[bi, pl.ds(CONT + f * ED, 32)] = acc
            return carry

        lax.fori_loop(0, NCAT, field, 0)

    def body(g, carry):
        b0 = base + g * GRP
        pltpu.sync_copy(ids_hbm.at[pl.ds(b0, GRP)], idx_g)
        fire(0, rows0, sem0)
        for bi in range(1, GRP + 1):
            if bi < GRP:
                fire(bi, (rows0, rows1)[bi % 2], (sem0, sem1)[bi % 2])
            wait(bi - 1, (rows0, rows1)[(bi - 1) % 2], (sem0, sem1)[(bi - 1) % 2])
            accum(bi - 1, (rows0, rows1)[(bi - 1) % 2])
        pltpu.sync_copy(out_v, v_hbm.at[pl.ds(b0, GRP)])
        return carry

    lax.fori_loop(0, NGRP, body, 0)


_pool = pl.kernel(
    _pool_body,
    out_type=jax.ShapeDtypeStruct((B, VPAD), jnp.bfloat16),
    mesh=plsc.VectorSubcoreMesh(
        core_axis_name="c", subcore_axis_name="s", num_cores=NC, num_subcores=NS
    ),
    scratch_types=[
        pltpu.VMEM((GRP, NCAT, T), jnp.int32),
        pltpu.VMEM((NIDS, ED), jnp.bfloat16),
        pltpu.VMEM((NIDS, ED), jnp.bfloat16),
        pltpu.VMEM((GRP, VPAD), jnp.bfloat16),
        pltpu.SemaphoreType.DMA,
        pltpu.SemaphoreType.DMA,
    ],
    compiler_params=pltpu.CompilerParams(use_tc_tiling_on_sc=False),
)

BB = 512


def _mix_body(z_ref, v_ref, c_ref, m_ref, wp_ref, wc_ref, bp_ref, wg_ref, bg_ref, o_ref):
    zb = z_ref[...]
    gate = jax.nn.sigmoid(
        jnp.dot(zb, wg_ref[...], preferred_element_type=jnp.float32) + bg_ref[...]
    )
    cpool = jnp.dot(c_ref[...], m_ref[...], preferred_element_type=jnp.float32)
    v32 = v_ref[...].astype(jnp.float32)
    exo = (
        jnp.dot(v32, wp_ref[...], preferred_element_type=jnp.float32)
        + jnp.dot(cpool, wc_ref[...], preferred_element_type=jnp.float32)
        + bp_ref[...]
    )
    o_ref[...] = zb + gate * exo


def _mix(z, v, cont2, m, wp, wc, bp, wg, bg):
    return pl.pallas_call(
        _mix_body,
        grid=(B // BB,),
        in_specs=[
            pl.BlockSpec((BB, ZD), lambda i: (i, 0)),
            pl.BlockSpec((BB, VPAD), lambda i: (i, 0)),
            pl.BlockSpec((BB, T * CONT), lambda i: (i, 0)),
            pl.BlockSpec((T * CONT, CONT), lambda i: (0, 0)),
            pl.BlockSpec((VPAD, ZD), lambda i: (0, 0)),
            pl.BlockSpec((CONT, ZD), lambda i: (0, 0)),
            pl.BlockSpec((1, ZD), lambda i: (0, 0)),
            pl.BlockSpec((ZD, ZD), lambda i: (0, 0)),
            pl.BlockSpec((1, ZD), lambda i: (0, 0)),
        ],
        out_specs=pl.BlockSpec((BB, ZD), lambda i: (i, 0)),
        out_shape=jax.ShapeDtypeStruct((B, ZD), jnp.float32),
    )(z, v, cont2, m, wp, wc, bp, wg, bg)


def kernel(z, past_exo_cont, past_exo_cat, tables, W_proj, b_proj, W_gate, b_gate):
    ids = jnp.clip(past_exo_cat, 0, V - 1).astype(jnp.int32)
    flat = jnp.transpose(ids, (0, 2, 1))
    tab = tables.astype(jnp.bfloat16)

    v = _pool(flat, tab)

    cont2 = past_exo_cont.reshape(B, T * CONT)
    m = jnp.tile(jnp.eye(CONT, dtype=jnp.float32), (T, 1)) * (1.0 / T)
    wp = jnp.concatenate(
        [jnp.zeros((CONT, ZD), W_proj.dtype), W_proj[CONT:] * (1.0 / T),
         jnp.zeros((VPAD - IN_DIM, ZD), W_proj.dtype)], axis=0
    )
    return _mix(
        z, v, cont2, m, wp, W_proj[:CONT],
        b_proj.reshape(1, ZD), W_gate, b_gate.reshape(1, ZD),
    )
